# Initial kernel scaffold; baseline (speedup 1.0000x reference)
import jax
import jax.numpy as jnp
from jax.experimental import pallas as pl


def kernel(embs, vis, masks, W, b):
    raise NotImplementedError("write your pallas kernel here")


# speedup vs baseline: 3.5329x; 3.5329x over previous
import functools

import jax
import jax.numpy as jnp
from jax.experimental import pallas as pl

_ALPHA = 0.9
_NUM_PARTS = 7
_FEATURE_DIM = 256


def _fused_kernel(vis_ref, mask_ref, embs_ref, w_ref, b_ref, out_ref, *,
                  T, V, D, FD, alpha):
    G = vis_ref.shape[0]
    f32 = jnp.float32

    v = jnp.zeros((G, V), f32)
    A = [None] * T
    C = [None] * T
    for t in range(T - 1, -1, -1):
        vis_t = vis_ref[:, t * V:(t + 1) * V]
        m = mask_ref[:, t:t + 1]
        v_nz = (v != 0.0).astype(f32)
        d_nz = (vis_t != 0.0).astype(f32)
        xor = v_nz + d_nz - 2.0 * v_nz * d_nz
        a_t = v * vis_t * alpha + xor * v
        c_t = v * vis_t * (1.0 - alpha) + xor * vis_t
        A[t] = m * a_t + (1.0 - m)
        C[t] = m * c_t
        v = m * jnp.maximum(v, vis_t) + (1.0 - m) * v

    part_row = jax.lax.broadcasted_iota(jnp.int32, (V, D), 0)
    part_col = jax.lax.broadcasted_iota(jnp.int32, (V, D), 1) // FD
    expand = (part_row == part_col).astype(f32)

    acc = jnp.zeros((G, D), f32)
    prod = jnp.ones((G, V), f32)
    for t in range(T):
        w_t = C[t] * prod
        prod = prod * A[t]
        w_full = jax.lax.dot_general(
            w_t, expand, (((1,), (0,)), ((), ())),
            preferred_element_type=f32)
        acc = acc + w_full * embs_ref[:, t * D:(t + 1) * D]

    lin = jax.lax.dot_general(
        acc, w_ref[:, :], (((1,), (1,)), ((), ())),
        preferred_element_type=f32)
    lin = lin + b_ref[:, :]
    new_mask = jnp.max(mask_ref[:, :], axis=1, keepdims=True)
    out_ref[:, :] = jnp.where(new_mask > 0.0, lin, 0.0)


def kernel(embs, vis, masks, W, b):
    B, N, T, D = embs.shape
    V = vis.shape[-1]
    K = W.shape[0]
    FD = D // _NUM_PARTS
    R = B * N
    G = 32

    embs2 = embs.reshape(R, T * D)
    vis2 = vis.reshape(R, T * V)
    masks2 = masks.reshape(R, T).astype(jnp.float32)
    b2 = b.reshape(1, K)

    body = functools.partial(_fused_kernel, T=T, V=V, D=D, FD=FD, alpha=_ALPHA)
    out = pl.pallas_call(
        body,
        grid=(R // G,),
        in_specs=[
            pl.BlockSpec((G, T * V), lambda i: (i, 0)),
            pl.BlockSpec((G, T), lambda i: (i, 0)),
            pl.BlockSpec((G, T * D), lambda i: (i, 0)),
            pl.BlockSpec((K, D), lambda i: (0, 0)),
            pl.BlockSpec((1, K), lambda i: (0, 0)),
        ],
        out_specs=pl.BlockSpec((G, K), lambda i: (i, 0)),
        out_shape=jax.ShapeDtypeStruct((R, K), jnp.float32),
    )(vis2, masks2, embs2, W, b2)
    return out.reshape(B, N, K)


# speedup vs baseline: 4.7084x; 1.3327x over previous
import functools

import jax
import jax.numpy as jnp
from jax.experimental import pallas as pl

_ALPHA = 0.9
_NUM_PARTS = 7
_FEATURE_DIM = 256


def _fused_kernel(vis_ref, mask_ref, embs_ref, w_ref, b_ref, out_ref, *,
                  T, V, D, FD, alpha):
    G = vis_ref.shape[0]
    f32 = jnp.float32

    v = jnp.zeros((G, V), f32)
    A = [None] * T
    C = [None] * T
    for t in range(T - 1, -1, -1):
        vis_t = vis_ref[:, t, :]
        m = mask_ref[:, t:t + 1]
        v_nz = (v != 0.0).astype(f32)
        d_nz = (vis_t != 0.0).astype(f32)
        xor = v_nz + d_nz - 2.0 * v_nz * d_nz
        a_t = v * vis_t * alpha + xor * v
        c_t = v * vis_t * (1.0 - alpha) + xor * vis_t
        A[t] = m * a_t + (1.0 - m)
        C[t] = m * c_t
        v = m * jnp.maximum(v, vis_t) + (1.0 - m) * v

    part_row = jax.lax.broadcasted_iota(jnp.int32, (V, D), 0)
    part_col = jax.lax.broadcasted_iota(jnp.int32, (V, D), 1) // FD
    expand = (part_row == part_col).astype(f32)

    acc = jnp.zeros((G, D), f32)
    prod = jnp.ones((G, V), f32)
    for t in range(T):
        w_t = C[t] * prod
        prod = prod * A[t]
        w_full = jax.lax.dot_general(
            w_t, expand, (((1,), (0,)), ((), ())),
            preferred_element_type=f32)
        acc = acc + w_full * embs_ref[:, t, :]

    lin = jax.lax.dot_general(
        acc, w_ref[:, :], (((1,), (1,)), ((), ())),
        preferred_element_type=f32)
    lin = lin + b_ref[:, :]
    new_mask = jnp.max(mask_ref[:, :], axis=1, keepdims=True)
    out_ref[:, :] = jnp.where(new_mask > 0.0, lin, 0.0)


def kernel(embs, vis, masks, W, b):
    B, N, T, D = embs.shape
    V = vis.shape[-1]
    K = W.shape[0]
    FD = D // _NUM_PARTS
    R = B * N
    G = 32

    embs2 = embs.reshape(R, T, D)
    vis2 = vis.reshape(R, T, V)
    masks2 = masks.reshape(R, T).astype(jnp.float32)
    b2 = b.reshape(1, K)

    body = functools.partial(_fused_kernel, T=T, V=V, D=D, FD=FD, alpha=_ALPHA)
    out = pl.pallas_call(
        body,
        grid=(R // G,),
        in_specs=[
            pl.BlockSpec((G, T, V), lambda i: (i, 0, 0)),
            pl.BlockSpec((G, T), lambda i: (i, 0)),
            pl.BlockSpec((G, T, D), lambda i: (i, 0, 0)),
            pl.BlockSpec((K, D), lambda i: (0, 0)),
            pl.BlockSpec((1, K), lambda i: (0, 0)),
        ],
        out_specs=pl.BlockSpec((G, K), lambda i: (i, 0)),
        out_shape=jax.ShapeDtypeStruct((R, K), jnp.float32),
    )(vis2, masks2, embs2, W, b2)
    return out.reshape(B, N, K)
